# baseline (speedup 1.0000x reference)
import functools

import jax
import jax.numpy as jnp
from jax import lax
from jax.experimental import pallas as pl
from jax.experimental.pallas import tpu as pltpu
from jax.experimental.pallas import tpu_sc as plsc

_N_FIELDS = 26
_VOCAB = 100000
_D = 32
_B = 16384

_NC = 2
_NS = 16
_NW = _NC * _NS
_HALF = _B // 2


_CCH = 4
_CW = _VOCAB // _CCH
_NH = 4
_Q = _B // _NH


def _sc_body(
    idx_hbm, tab_hbm, out_hbm, idx_v, col_v, out_v, idx_sh, csem, isem, osem
):
    sid = lax.axis_index("s")
    wid = sid * _NC + lax.axis_index("c")

    def col_copy(f):
        return [pltpu.make_async_copy(tab_hbm.at[f * _D + wid], col_v, csem)]

    def idx_copy(f, h, slot):
        return pltpu.make_async_copy(
            idx_hbm.at[pl.ds(f * _B + h * _Q, _Q)], idx_sh.at[slot], isem
        )

    def out_copy(f, h, slot):
        return pltpu.make_async_copy(
            out_v.at[slot], out_hbm.at[f * _D + wid, pl.ds(h * _Q, _Q)], osem
        )

    for c in col_copy(0):
        c.start()

    @pl.when(sid == 0)
    def _():
        idx_copy(0, 0, 0).start()

    def field_body(f, carry):
        for c in col_copy(f):
            c.wait()

        for h in range(_NH):
            slot = h % 2

            @pl.when(sid == 0)
            def _(f=f, h=h, slot=slot):
                idx_copy(f, h, slot).wait()

            plsc.subcore_barrier()

            @pl.when(sid == 0)
            def _(f=f, h=h, slot=slot):
                if h < _NH - 1:
                    idx_copy(f, h + 1, 1 - slot).start()
                else:

                    @pl.when(f + 1 < _N_FIELDS)
                    def _():
                        idx_copy(f + 1, 0, 1 - slot).start()

            pltpu.sync_copy(idx_sh.at[slot], idx_v.at[slot])

            if h >= 2:
                out_copy(f, h, slot).wait()
            else:

                @pl.when(f >= 1)
                def _(f=f, h=h, slot=slot):
                    out_copy(f, h, slot).wait()

            @plsc.parallel_loop(0, _Q, step=16, unroll=16)
            def _(i, slot=slot):
                sl = pl.ds(i, 16)
                out_v[slot, sl] = plsc.load_gather(col_v, [idx_v[slot, sl]])

            if h == _NH - 1:
                @pl.when(f + 1 < _N_FIELDS)
                def _(f=f):
                    for c in col_copy(f + 1):
                        c.start()

            out_copy(f, h, slot).start()
        return carry

    lax.fori_loop(0, _N_FIELDS, field_body, 0)

    out_copy(_N_FIELDS - 1, _NH - 2, 0).wait()
    out_copy(_N_FIELDS - 1, _NH - 1, 1).wait()


@functools.partial(jax.jit, static_argnames=())
def kernel(inputs, tables):
    idxs = inputs.reshape(_N_FIELDS * _B)
    tab_t = jnp.transpose(tables, (0, 2, 1))
    tab2 = tab_t.reshape(_N_FIELDS * _D, _VOCAB)

    mesh = plsc.VectorSubcoreMesh(core_axis_name="c", subcore_axis_name="s")
    run = pl.kernel(
        _sc_body,
        out_type=jax.ShapeDtypeStruct((_N_FIELDS * _D, _B), jnp.float32),
        mesh=mesh,
        scratch_types=[
            pltpu.VMEM((2, _Q), jnp.int32),
            pltpu.VMEM((_VOCAB,), jnp.float32),
            pltpu.VMEM((2, _Q), jnp.float32),
            pltpu.VMEM_SHARED((2, _Q), jnp.int32),
            pltpu.SemaphoreType.DMA,
            pltpu.SemaphoreType.DMA,
            pltpu.SemaphoreType.DMA,
        ],
        compiler_params=pltpu.CompilerParams(needs_layout_passes=False),
    )
    out_t = run(idxs, tab2)
    return out_t.reshape(_N_FIELDS, _D, _B).transpose(0, 2, 1)


# speedup vs baseline: 1.0985x; 1.0985x over previous
import functools

import jax
import jax.numpy as jnp
from jax import lax
from jax.experimental import pallas as pl
from jax.experimental.pallas import tpu as pltpu
from jax.experimental.pallas import tpu_sc as plsc

_N_FIELDS = 26
_VOCAB = 100000
_D = 32
_B = 16384

_NC = 2
_NS = 16
_NW = _NC * _NS
_HALF = _B // 2


_CCH = 4
_CW = _VOCAB // _CCH
_NH = 4
_Q = _B // _NH


def _sc_body(
    idx_hbm, tab_hbm, out_hbm, idx_v, col_v, out_v, idx_sh,
    csem, isem, osem, psem,
):
    sid = lax.axis_index("s")
    wid = sid * _NC + lax.axis_index("c")

    def col_copy(f):
        return [pltpu.make_async_copy(tab_hbm.at[f * _D + wid], col_v, csem)]

    def idx_copy(f, fslot):
        return pltpu.make_async_copy(
            idx_hbm.at[pl.ds(f * _B, _B)], idx_sh.at[fslot], isem
        )

    def idx_pull(fslot, h, slot):
        return pltpu.make_async_copy(
            idx_sh.at[fslot, pl.ds(h * _Q, _Q)], idx_v.at[slot], psem
        )

    def out_copy(f, h, slot):
        return pltpu.make_async_copy(
            out_v.at[slot], out_hbm.at[f * _D + wid, pl.ds(h * _Q, _Q)], osem
        )

    for c in col_copy(0):
        c.start()

    @pl.when(sid == 0)
    def _():
        idx_copy(0, 0).start()

    def field_body(f, carry):
        fslot = f % 2

        for c in col_copy(f):
            c.wait()

        @pl.when(sid == 0)
        def _(f=f, fslot=fslot):
            idx_copy(f, fslot).wait()

        plsc.subcore_barrier()

        @pl.when(sid == 0)
        def _(f=f, fslot=fslot):
            @pl.when(f + 1 < _N_FIELDS)
            def _():
                idx_copy(f + 1, 1 - fslot).start()

        idx_pull(fslot, 0, 0).start()

        for h in range(_NH):
            slot = h % 2
            if h < _NH - 1:
                idx_pull(fslot, h + 1, 1 - slot).start()
            idx_pull(fslot, h, slot).wait()

            if h >= 2:
                out_copy(f, h, slot).wait()
            else:

                @pl.when(f >= 1)
                def _(f=f, h=h, slot=slot):
                    out_copy(f, h, slot).wait()

            @plsc.parallel_loop(0, _Q, step=16, unroll=16)
            def _(i, slot=slot):
                sl = pl.ds(i, 16)
                out_v[slot, sl] = plsc.load_gather(col_v, [idx_v[slot, sl]])

            if h == _NH - 1:
                @pl.when(f + 1 < _N_FIELDS)
                def _(f=f):
                    for c in col_copy(f + 1):
                        c.start()

            out_copy(f, h, slot).start()
        return carry

    lax.fori_loop(0, _N_FIELDS, field_body, 0)

    out_copy(_N_FIELDS - 1, _NH - 2, 0).wait()
    out_copy(_N_FIELDS - 1, _NH - 1, 1).wait()


@functools.partial(jax.jit, static_argnames=())
def kernel(inputs, tables):
    idxs = inputs.reshape(_N_FIELDS * _B)
    tab_t = jnp.transpose(tables, (0, 2, 1))
    tab2 = tab_t.reshape(_N_FIELDS * _D, _VOCAB)

    mesh = plsc.VectorSubcoreMesh(core_axis_name="c", subcore_axis_name="s")
    run = pl.kernel(
        _sc_body,
        out_type=jax.ShapeDtypeStruct((_N_FIELDS * _D, _B), jnp.float32),
        mesh=mesh,
        scratch_types=[
            pltpu.VMEM((2, _Q), jnp.int32),
            pltpu.VMEM((_VOCAB,), jnp.float32),
            pltpu.VMEM((2, _Q), jnp.float32),
            pltpu.VMEM_SHARED((2, _B), jnp.int32),
            pltpu.SemaphoreType.DMA,
            pltpu.SemaphoreType.DMA,
            pltpu.SemaphoreType.DMA,
            pltpu.SemaphoreType.DMA,
        ],
        compiler_params=pltpu.CompilerParams(needs_layout_passes=False),
    )
    out_t = run(idxs, tab2)
    return out_t.reshape(_N_FIELDS, _D, _B).transpose(0, 2, 1)


# speedup vs baseline: 1.1559x; 1.0522x over previous
import functools

import jax
import jax.numpy as jnp
from jax import lax
from jax.experimental import pallas as pl
from jax.experimental.pallas import tpu as pltpu
from jax.experimental.pallas import tpu_sc as plsc

_N_FIELDS = 26
_VOCAB = 100000
_D = 32
_B = 16384

_NC = 2
_NS = 16
_NW = _NC * _NS
_HALF = _B // 2


_CCH = 4
_CW = _VOCAB // _CCH
_NH = 4
_Q = _B // _NH


def _sc_body(
    idx_hbm, tab_hbm, out_hbm, idx_v, col_v, out_v, idx_sh,
    csem, isem, osem, psem,
):
    sid = lax.axis_index("s")
    wid = sid * _NC + lax.axis_index("c")

    def col_copy(f):
        return [pltpu.make_async_copy(tab_hbm.at[f * _D + wid], col_v, csem)]

    def idx_copy(f, fslot):
        return pltpu.make_async_copy(
            idx_hbm.at[pl.ds(f * _B, _B)], idx_sh.at[fslot], isem
        )

    def idx_pull(fslot, h, slot):
        return pltpu.make_async_copy(
            idx_sh.at[fslot, pl.ds(h * _Q, _Q)], idx_v.at[slot], psem
        )

    def out_copy(f, h, slot):
        return pltpu.make_async_copy(
            out_v.at[slot], out_hbm.at[f * _D + wid, pl.ds(h * _Q, _Q)], osem
        )

    for c in col_copy(0):
        c.start()

    @pl.when(sid == 0)
    def _():
        idx_copy(0, 0).start()
        idx_copy(0, 0).wait()

    plsc.subcore_barrier()

    @pl.when(sid == 0)
    def _():
        idx_copy(1, 1).start()

    idx_pull(0, 0, 0).start()

    def field_body(f, carry):
        fslot = f % 2

        for c in col_copy(f):
            c.wait()

        for h in range(_NH):
            slot = h % 2
            if h < _NH - 1:
                idx_pull(fslot, h + 1, 1 - slot).start()
            idx_pull(fslot, h, slot).wait()

            if h >= 2:
                out_copy(f, h, slot).wait()
            else:

                @pl.when(f >= 1)
                def _(f=f, h=h, slot=slot):
                    out_copy(f, h, slot).wait()

            @plsc.parallel_loop(0, _Q, step=16, unroll=16)
            def _(i, slot=slot):
                sl = pl.ds(i, 16)
                out_v[slot, sl] = plsc.load_gather(col_v, [idx_v[slot, sl]])

            if h == _NH - 1:
                @pl.when(f + 1 < _N_FIELDS)
                def _(f=f):
                    for c in col_copy(f + 1):
                        c.start()

            out_copy(f, h, slot).start()

        @pl.when(jnp.logical_and(sid == 0, f + 1 < _N_FIELDS))
        def _(f=f, fslot=fslot):
            idx_copy(f + 1, 1 - fslot).wait()

        plsc.subcore_barrier()

        @pl.when(jnp.logical_and(sid == 0, f + 2 < _N_FIELDS))
        def _(f=f, fslot=fslot):
            idx_copy(f + 2, fslot).start()

        @pl.when(f + 1 < _N_FIELDS)
        def _(fslot=fslot):
            idx_pull(1 - fslot, 0, 0).start()

        return carry

    lax.fori_loop(0, _N_FIELDS, field_body, 0)

    out_copy(_N_FIELDS - 1, _NH - 2, 0).wait()
    out_copy(_N_FIELDS - 1, _NH - 1, 1).wait()


@functools.partial(jax.jit, static_argnames=())
def kernel(inputs, tables):
    idxs = inputs.reshape(_N_FIELDS * _B)
    tab_t = jnp.transpose(tables, (0, 2, 1))
    tab2 = tab_t.reshape(_N_FIELDS * _D, _VOCAB)

    mesh = plsc.VectorSubcoreMesh(core_axis_name="c", subcore_axis_name="s")
    run = pl.kernel(
        _sc_body,
        out_type=jax.ShapeDtypeStruct((_N_FIELDS * _D, _B), jnp.float32),
        mesh=mesh,
        scratch_types=[
            pltpu.VMEM((2, _Q), jnp.int32),
            pltpu.VMEM((_VOCAB,), jnp.float32),
            pltpu.VMEM((2, _Q), jnp.float32),
            pltpu.VMEM_SHARED((2, _B), jnp.int32),
            pltpu.SemaphoreType.DMA,
            pltpu.SemaphoreType.DMA,
            pltpu.SemaphoreType.DMA,
            pltpu.SemaphoreType.DMA,
        ],
        compiler_params=pltpu.CompilerParams(needs_layout_passes=False),
    )
    out_t = run(idxs, tab2)
    return out_t.reshape(_N_FIELDS, _D, _B).transpose(0, 2, 1)
